# Initial kernel scaffold; baseline (speedup 1.0000x reference)
import jax
import jax.numpy as jnp
from jax.experimental import pallas as pl


def kernel(inputs):
    raise NotImplementedError("write your pallas kernel here")


# speedup vs baseline: 15.4042x; 15.4042x over previous
import functools

import jax
import jax.numpy as jnp
import numpy as np
from jax import lax
from jax.experimental import pallas as pl
from jax.experimental.pallas import tpu as pltpu
from jax.experimental.pallas import tpu_sc as plsc

B = 128
T = 4
N = 32768
L = 16
NC = 2
NS = 16
NW = NC * NS
ROWS_PER_W = B // NW
CHUNKS = N // L

_INF = np.float32(np.inf)
_BIGI = np.int32(2**31 - 1)


_GDN = lax.GatherDimensionNumbers(
    offset_dims=(), collapsed_slice_dims=(0,), start_index_map=(0,))


def _perm(x, idx):
    return lax.gather(x, idx.reshape(L, 1), _GDN, slice_sizes=(1,),
                      mode=lax.GatherScatterMode.PROMISE_IN_BOUNDS)


def _bfly_min(v, lane):
    for k in (1, 2, 4, 8):
        v = jnp.minimum(v, _perm(v, lane ^ k))
    return v


def _lex_argmin(vals, idxs, lane):
    mn = jnp.minimum(jnp.minimum(vals[0], vals[1]),
                     jnp.minimum(vals[2], vals[3]))
    s = _bfly_min(mn, lane)
    cand = [jnp.where(vals[j] == s, idxs[j], _BIGI) for j in range(4)]
    cm = jnp.minimum(jnp.minimum(cand[0], cand[1]),
                     jnp.minimum(cand[2], cand[3]))
    si = _bfly_min(cm, lane)
    return s, si


def _log2_1p(s):
    m = jnp.float32(1.0) + s
    sp = m - jnp.float32(1.0)
    z = sp / (jnp.float32(2.0) + sp)
    z2 = z * z
    p = z * (jnp.float32(1.0)
             + z2 * (jnp.float32(1.0 / 3.0)
                     + z2 * (jnp.float32(1.0 / 5.0)
                             + z2 * (jnp.float32(1.0 / 7.0)
                                     + z2 * jnp.float32(1.0 / 9.0)))))
    return jnp.float32(2.8853900817779268) * p


@jax.jit
def _sc_kth_best(x):
    mesh = plsc.VectorSubcoreMesh(core_axis_name="c", subcore_axis_name="s")

    @functools.partial(
        pl.kernel,
        mesh=mesh,
        out_type=[
            jax.ShapeDtypeStruct((NW, L), jnp.float32),
            jax.ShapeDtypeStruct((NW, L), jnp.int32),
        ],
        scratch_types=[
            pltpu.VMEM((N,), jnp.float32),
            pltpu.VMEM((L,), jnp.float32),
            pltpu.VMEM((L,), jnp.int32),
        ],
    )
    def k(x_hbm, rate_hbm, idx_hbm, buf, rate_v, idx_v):
        cid = lax.axis_index("c")
        sid = lax.axis_index("s")
        wid = sid * NC + cid
        lane = lax.iota(jnp.int32, L)

        rate_acc = jnp.zeros((L,), jnp.float32)
        idx_acc = jnp.zeros((L,), jnp.int32)

        for r in range(ROWS_PER_W):
            row = wid * ROWS_PER_W + r
            pltpu.sync_copy(x_hbm.at[row * T + (T - 1)], buf)

            def body(c, carry):
                m1, m2, m3, m4, i1, i2, i3, i4 = carry
                off = pl.multiple_of(c * L, 8)
                v = buf[pl.ds(off, L)]
                idxv = lane + c * L
                c1 = v < m1
                c2 = v < m2
                c3 = v < m3
                c4 = v < m4
                nm4 = jnp.where(c4, jnp.where(c3, m3, v), m4)
                ni4 = jnp.where(c4, jnp.where(c3, i3, idxv), i4)
                nm3 = jnp.where(c3, jnp.where(c2, m2, v), m3)
                ni3 = jnp.where(c3, jnp.where(c2, i2, idxv), i3)
                nm2 = jnp.where(c2, jnp.where(c1, m1, v), m2)
                ni2 = jnp.where(c2, jnp.where(c1, i1, idxv), i2)
                nm1 = jnp.where(c1, v, m1)
                ni1 = jnp.where(c1, idxv, i1)
                return (nm1, nm2, nm3, nm4, ni1, ni2, ni3, ni4)

            init = (jnp.full((L,), _INF),) * 4 + (jnp.full((L,), _BIGI),) * 4
            m1, m2, m3, m4, i1, i2, i3, i4 = lax.fori_loop(
                0, CHUNKS, body, init)

            vals = [m1, m2, m3, m4]
            idxs = [i1, i2, i3, i4]
            for _ in range(3):
                s, si = _lex_argmin(vals, idxs, lane)
                vals = [jnp.where((vals[j] == s) & (idxs[j] == si), _INF,
                                  vals[j]) for j in range(4)]
            s, si = _lex_argmin(vals, idxs, lane)

            rate = jnp.float32(0.9) * _log2_1p(s)
            rate_acc = jnp.where(lane == r, rate, rate_acc)
            idx_acc = jnp.where(lane == r, si, idx_acc)

        rate_v[...] = rate_acc
        idx_v[...] = idx_acc
        pltpu.sync_copy(rate_v, rate_hbm.at[wid])
        pltpu.sync_copy(idx_v, idx_hbm.at[wid])

    return k(x)


def kernel(inputs):
    x = inputs.reshape(B * T, N)
    rate2, idx2 = _sc_kth_best(x)
    rate = rate2[:, :ROWS_PER_W].reshape(B)
    idx = idx2[:, :ROWS_PER_W].reshape(B)
    return (rate, idx)
